# baseline (speedup 1.0000x reference)
import functools

import jax
import jax.numpy as jnp
from jax import lax
from jax.experimental import pallas as pl
from jax.experimental.pallas import tpu as pltpu
from jax.experimental.pallas import tpu_sc as plsc

N = 10000
E_RAW = 320000
E = E_RAW + N
D = 128
H1, C1 = 8, 8
F1 = H1 * C1
C2 = 16
G = 64

NW = 32
BLK = 128
ZB = 128
EP = 331776
PT = EP // NW
NP = 10016
AR = 10240
RPT = AR // 16

_P = jax.lax.Precision.HIGHEST

_GDN = lax.GatherDimensionNumbers(
    offset_dims=(), collapsed_slice_dims=(0,), start_index_map=(0,))


def _vperm(v, idx):
    return lax.gather(v, idx[:, None], _GDN, slice_sizes=(1,),
                      mode=lax.GatherScatterMode.PROMISE_IN_BOUNDS)


@functools.lru_cache(maxsize=1)
def _sc_mesh():
    return plsc.VectorSubcoreMesh(core_axis_name="c", subcore_axis_name="s")


def _f32(*shape):
    return jax.ShapeDtypeStruct(shape, jnp.float32)


def _proj_body(x_ref, w1_ref, w2_ref, t1_ref, t2_ref):
    x = x_ref[...]
    t1_ref[...] = jnp.dot(x, w1_ref[...], precision=_P)
    t2_ref[...] = jnp.dot(x, w2_ref[...], precision=_P)


def _proj(x, w1, w2):
    n, f = x.shape[0], w1.shape[1]
    return pl.pallas_call(
        _proj_body,
        out_shape=[_f32(n, f), _f32(n, f)],
    )(x, w1, w2)


def _norm1_body(acc_ref, rep_ref, b1_ref, wl2_ref, wr2_ref, t1_ref, t2_ref):
    s = acc_ref[0] + acc_ref[1]
    num = s[:, :F1]
    den = s[:, F1:F1 + H1]
    den_f = jnp.dot(den, rep_ref[...], precision=_P)
    h = num / (den_f + 1e-16) + b1_ref[...]
    h = jnp.where(h > 0, h, jnp.exp(jnp.minimum(h, 0.0)) - 1.0)
    mask = (lax.broadcasted_iota(jnp.int32, (AR, 1), 0) < N).astype(jnp.float32)
    h = h * mask
    t1_ref[...] = jnp.dot(h, wl2_ref[...], precision=_P)
    t2_ref[...] = jnp.dot(h, wr2_ref[...], precision=_P)


def _norm1(acc, rep, b1, wl2, wr2):
    return pl.pallas_call(
        _norm1_body,
        out_shape=[_f32(AR, C2), _f32(AR, C2)],
    )(acc, rep, b1.reshape(1, F1), wl2, wr2)


def _final_body(acc_ref, b2_ref, batch_ref, wc_ref, bc_ref,
                pooled_ref, out_ref):
    s = acc_ref[0] + acc_ref[1]
    num = s[:, :C2]
    den = s[:, C2:C2 + 1]
    h2 = num / (den + 1e-16) + b2_ref[...]
    h2 = h2[:N, :]
    gids = lax.broadcasted_iota(jnp.int32, (N, G), 1)
    onehot = (batch_ref[...] == gids).astype(jnp.float32)
    sums = lax.dot_general(onehot, h2, (((0,), (0,)), ((), ())),
                           precision=_P)
    counts = jnp.sum(onehot, axis=0)[:, None]
    pooled = sums / jnp.maximum(counts, 1.0)
    pooled_ref[...] = pooled
    out_ref[...] = jnp.dot(pooled, wc_ref[...], precision=_P) + bc_ref[...]


def _final(acc, b2, batch, wc, bc):
    return pl.pallas_call(
        _final_body,
        out_shape=[_f32(G, C2), _f32(G, 1)],
    )(acc, b2.reshape(1, C2), batch.reshape(N, 1), wc, bc.reshape(1, 1))


def _sc_wid():
    return lax.axis_index("s") * 2 + lax.axis_index("c")


def _edge_body(nv, grp, t1_hbm, t2_hbm, src_hbm, dst_hbm, att_hbm, z_hbm,
               oacc_hbm, idxs_v, idxd_v, bufa_v, bufb_v, pay_v, att_v,
               acc_s, sema, semb):
    cid = lax.axis_index("c")
    sid = lax.axis_index("s")
    base = _sc_wid() * PT
    r0 = sid * RPT

    iota = lax.iota(jnp.int32, 16)
    selp = (iota & 1) * 8
    zeros16 = jnp.zeros((16,), jnp.float32)
    lane0 = (iota == 0).astype(jnp.float32)
    pair_masks = [((iota >> 1) == j).astype(jnp.float32) for j in range(nv)]

    pltpu.sync_copy(att_hbm, att_v)
    attv = [att_v[pl.ds(16 * j, 16)] for j in range(nv)]

    @pl.loop(0, RPT, step=ZB)
    def _(j):
        pltpu.sync_copy(z_hbm, acc_s.at[pl.ds(r0 + j, ZB)])

    plsc.subcore_barrier()

    @pl.loop(0, PT, step=BLK)
    def _(off):
        b = base + off
        pltpu.sync_copy(src_hbm.at[pl.ds(b, BLK)], idxs_v)
        pltpu.sync_copy(dst_hbm.at[pl.ds(b, BLK)], idxd_v)
        cpa = pltpu.async_copy(t1_hbm.at[idxs_v], bufa_v, sema)
        cpb = pltpu.async_copy(t2_hbm.at[idxd_v], bufb_v, semb)
        cpa.wait()
        cpb.wait()

        @pl.loop(0, BLK)
        def _(r):
            ees = []
            for j in range(nv):
                a = bufa_v[r, pl.ds(16 * j, 16)]
                bb = bufb_v[r, pl.ds(16 * j, 16)]
                s = a + bb
                m = jnp.maximum(s, 0.2 * s)
                p = m * attv[j]
                k = 1
                while k < grp:
                    p = p + _vperm(p, iota ^ k)
                    k *= 2
                ee = jnp.exp(p)
                pay_v[r, pl.ds(16 * j, 16)] = a * ee
                ees.append(ee)
            if grp == 8:
                den = zeros16
                for j in range(nv):
                    den = den + _vperm(ees[j], selp) * pair_masks[j]
            else:
                den = ees[0] * lane0
            pay_v[r, pl.ds(16 * nv, 16)] = den

        pltpu.sync_copy(pay_v, acc_s.at[idxd_v], add=True)

    plsc.subcore_barrier()
    pltpu.sync_copy(acc_s.at[pl.ds(r0, RPT)], oacc_hbm.at[cid, pl.ds(r0, RPT)])


def _sc_edge(t1, t2, src, dst, att_row, z, nv, grp):
    fw = 16 * nv
    pw = fw + 16
    k = pl.kernel(
        functools.partial(_edge_body, nv, grp),
        out_type=[_f32(2, AR, pw)],
        mesh=_sc_mesh(),
        compiler_params=pltpu.CompilerParams(
            needs_layout_passes=False, use_tc_tiling_on_sc=False),
        scratch_types=[
            pltpu.VMEM((BLK,), jnp.int32),
            pltpu.VMEM((BLK,), jnp.int32),
            pltpu.VMEM((BLK, fw), jnp.float32),
            pltpu.VMEM((BLK, fw), jnp.float32),
            pltpu.VMEM((BLK, pw), jnp.float32),
            pltpu.VMEM((fw,), jnp.float32),
            pltpu.VMEM_SHARED((AR, pw), jnp.float32),
            pltpu.SemaphoreType.DMA,
            pltpu.SemaphoreType.DMA,
        ],
    )
    return k(t1, t2, src, dst, att_row, z)[0]


def kernel(x, edge_index, batch, W_l1, W_r1, att1, b1, W_l2, W_r2, att2, b2,
           Wc, bc):
    loops = jnp.arange(N, dtype=edge_index.dtype)
    pad = jnp.full((EP - E,), N, dtype=edge_index.dtype)
    src = jnp.concatenate([edge_index[0], loops, pad])
    dst = jnp.concatenate([edge_index[1], loops, pad])

    xpad = jnp.zeros((NP, D), jnp.float32).at[:N].set(x)

    att1_row = att1.reshape(F1)
    att2_row = att2.reshape(C2)

    heads = jnp.arange(H1)
    rep1 = (jnp.arange(F1)[None, :] // C1 ==
            heads[:, None]).astype(jnp.float32)

    z80 = jnp.zeros((ZB, F1 + 16), jnp.float32)
    z32 = jnp.zeros((ZB, C2 + 16), jnp.float32)

    t1, t2 = _proj(xpad, W_l1, W_r1)
    acc1 = _sc_edge(t1, t2, src, dst, att1_row, z80, 4, 8)
    t12, t22 = _norm1(acc1, rep1, b1, W_l2, W_r2)

    acc2 = _sc_edge(t12, t22, src, dst, att2_row, z32, 1, 16)

    pooled, out = _final(acc2, b2, batch, Wc, bc)
    return (out.reshape(-1), pooled)
